# Initial kernel scaffold; baseline (speedup 1.0000x reference)
import jax
import jax.numpy as jnp
from jax.experimental import pallas as pl


def kernel(t, pos_embeddings):
    raise NotImplementedError("write your pallas kernel here")


# speedup vs baseline: 1.7991x; 1.7991x over previous
import functools

import jax
import jax.numpy as jnp
from jax import lax
from jax.experimental import pallas as pl
from jax.experimental.pallas import tpu as pltpu
from jax.experimental.pallas import tpu_sc as plsc

MAX_T = 1000
D = 512
B = 16384

_info = plsc.get_sparse_core_info()
NC, NS = _info.num_cores, _info.num_subcores
NW = NC * NS
BPW = B // NW
CH = 64
NCH = BPW // CH


def _make_lookup():
    mesh = plsc.VectorSubcoreMesh(core_axis_name="c", subcore_axis_name="s")

    @functools.partial(
        pl.kernel,
        mesh=mesh,
        out_type=jax.ShapeDtypeStruct((B, D), jnp.float32),
        scratch_types=[
            pltpu.VMEM((BPW,), jnp.int32),
            pltpu.VMEM((2, CH, D), jnp.float32),
            pltpu.SemaphoreType.DMA,
            pltpu.SemaphoreType.DMA,
        ],
    )
    def lookup(t_hbm, table_hbm, out_hbm, idx_v, rows_v, sem0, sem1):
        wid = lax.axis_index("s") * NC + lax.axis_index("c")
        base = wid * BPW
        pltpu.sync_copy(t_hbm.at[pl.ds(base, BPW)], idx_v)
        sems = (sem0, sem1)
        copies = [None, None]
        copies[0] = pltpu.async_copy(
            table_hbm.at[idx_v.at[pl.ds(0, CH)]], rows_v.at[0], sems[0])
        for j in range(NCH):
            cur = j % 2
            if j + 1 < NCH:
                nxt = (j + 1) % 2
                copies[nxt] = pltpu.async_copy(
                    table_hbm.at[idx_v.at[pl.ds((j + 1) * CH, CH)]],
                    rows_v.at[nxt], sems[nxt])
            copies[cur].wait()
            pltpu.sync_copy(rows_v.at[cur],
                            out_hbm.at[pl.ds(base + j * CH, CH)])

    return lookup


_lookup = _make_lookup()


def kernel(t, pos_embeddings):
    return _lookup(t.astype(jnp.int32), pos_embeddings)
